# Initial kernel scaffold; baseline (speedup 1.0000x reference)
import jax
import jax.numpy as jnp
from jax.experimental import pallas as pl


def kernel(v_fea, t_fea, e_fea, ve, a_v_w, a_t_w, a_e_w):
    raise NotImplementedError("write your pallas kernel here")


# speedup vs baseline: 21.8822x; 21.8822x over previous
import functools

import jax
import jax.numpy as jnp
from jax import lax
from jax.experimental import pallas as pl
from jax.experimental.pallas import tpu as pltpu
from jax.experimental.pallas import tpu_sc as plsc

V_NUM = 10000
E_NUM = 10000
N_INC = 320000
D = 128
L = 16
NC = 2
NS = 16
NW = NC * NS
BATCH = 128
NBATCH = N_INC // BATCH
VT = 632
V_PAD = VT * NS

_MESH = plsc.VectorSubcoreMesh(
    core_axis_name="c", subcore_axis_name="s", num_cores=NC, num_subcores=NS)
_SC_PARAMS = pltpu.CompilerParams(needs_layout_passes=False)


def _scores_body(v_ref, t_ref, e_ref, wv_ref, wt_ref, we_ref, s_ref, q_ref):
    s_ref[...] = (
        jnp.dot(v_ref[...], wv_ref[...], preferred_element_type=jnp.float32)
        + jnp.dot(t_ref[...], wt_ref[...], preferred_element_type=jnp.float32))
    q_ref[...] = jnp.dot(e_ref[...], we_ref[...],
                         preferred_element_type=jnp.float32)


_scores = pl.pallas_call(
    _scores_body,
    out_shape=(jax.ShapeDtypeStruct((V_NUM, 1), jnp.float32),
               jax.ShapeDtypeStruct((E_NUM, 1), jnp.float32)))


def _edge_weights(s_loc, q_loc, vi, ei):
    x = plsc.load_gather(s_loc, [vi]) + plsc.load_gather(q_loc, [ei])
    t = jnp.exp(x + x)
    th = 1.0 - 2.0 / (t + 1.0)
    return jnp.exp(th + th)


@functools.partial(
    pl.kernel,
    out_type=(jax.ShapeDtypeStruct((V_PAD,), jnp.float32),
              jax.ShapeDtypeStruct((V_PAD,), jnp.float32)),
    mesh=_MESH,
    compiler_params=_SC_PARAMS,
    scratch_types=[
        pltpu.VMEM((V_NUM,), jnp.float32),
        pltpu.VMEM((E_NUM,), jnp.float32),
        pltpu.VMEM((BATCH,), jnp.int32),
        pltpu.VMEM((BATCH,), jnp.int32),
        pltpu.VMEM((BATCH,), jnp.float32),
        pltpu.VMEM((640,), jnp.float32),
        pltpu.VMEM_SHARED((V_PAD,), jnp.float32),
    ],
)
def _denom_kernel(s_hbm, q_hbm, v_hbm, e_hbm, den0_hbm, den1_hbm,
                  s_loc, q_loc, vbuf, ebuf, wbuf, zbuf, dacc):
    cid = lax.axis_index("c")
    sid = lax.axis_index("s")
    wid = cid * NS + sid
    pltpu.sync_copy(s_hbm, s_loc)
    pltpu.sync_copy(q_hbm, q_loc)

    def zbody(i, carry):
        zbuf[pl.ds(i * L, L)] = jnp.zeros((L,), jnp.float32)
        return carry

    lax.fori_loop(0, 640 // L, zbody, 0)
    pltpu.sync_copy(zbuf.at[pl.ds(0, VT)], dacc.at[pl.ds(sid * VT, VT)])
    plsc.subcore_barrier()

    nb = (NBATCH - wid + NW - 1) // NW

    def body(i, carry):
        base = (wid + i * NW) * BATCH
        pltpu.sync_copy(v_hbm.at[pl.ds(base, BATCH)], vbuf)
        pltpu.sync_copy(e_hbm.at[pl.ds(base, BATCH)], ebuf)
        for j in range(BATCH // L):
            vi = vbuf[pl.ds(j * L, L)]
            ei = ebuf[pl.ds(j * L, L)]
            wbuf[pl.ds(j * L, L)] = _edge_weights(s_loc, q_loc, vi, ei)
        pltpu.sync_copy(wbuf, dacc.at[vbuf], add=True)
        return carry

    lax.fori_loop(0, nb, body, 0)
    plsc.subcore_barrier()

    pltpu.sync_copy(dacc.at[pl.ds(sid * VT, VT)], zbuf.at[pl.ds(0, VT)])

    @pl.when(cid == 0)
    def _():
        pltpu.sync_copy(zbuf.at[pl.ds(0, VT)], den0_hbm.at[pl.ds(sid * VT, VT)])

    @pl.when(cid == 1)
    def _():
        pltpu.sync_copy(zbuf.at[pl.ds(0, VT)], den1_hbm.at[pl.ds(sid * VT, VT)])


_VT_CHUNKS = ((0, 128), (128, 128), (256, 128), (384, 128), (512, 120))


@functools.partial(
    pl.kernel,
    out_type=jax.ShapeDtypeStruct((NC, V_PAD, D), jnp.float32),
    mesh=_MESH,
    compiler_params=_SC_PARAMS,
    scratch_types=[
        pltpu.VMEM((V_NUM,), jnp.float32),
        pltpu.VMEM((E_NUM,), jnp.float32),
        pltpu.VMEM((V_PAD,), jnp.float32),
        pltpu.VMEM((BATCH,), jnp.int32),
        pltpu.VMEM((BATCH,), jnp.int32),
        pltpu.VMEM((BATCH,), jnp.float32),
        pltpu.VMEM((BATCH, D), jnp.float32),
        pltpu.VMEM_SHARED((V_PAD, D), jnp.float32),
        pltpu.SemaphoreType.DMA,
    ],
)
def _agg_kernel(s_hbm, q_hbm, v_hbm, e_hbm, den0_hbm, den1_hbm, efea_hbm,
                part_hbm,
                s_loc, q_loc, rden, vbuf, ebuf, cbuf, rows, acc, sem):
    cid = lax.axis_index("c")
    sid = lax.axis_index("s")
    wid = cid * NS + sid
    pltpu.sync_copy(s_hbm, s_loc)
    pltpu.sync_copy(q_hbm, q_loc)
    pltpu.sync_copy(den0_hbm, rden)

    def rbody(i, carry):
        pltpu.sync_copy(den1_hbm.at[pl.ds(i * BATCH, BATCH)], cbuf)
        for j in range(BATCH // L):
            sl = pl.ds(i * BATCH + j * L, L)
            rden[sl] = 1.0 / (rden[sl] + cbuf[pl.ds(j * L, L)])
        return carry

    lax.fori_loop(0, V_PAD // BATCH, rbody, 0)

    def zbody(r, carry):
        for ch in range(D // L):
            rows[r, pl.ds(ch * L, L)] = jnp.zeros((L,), jnp.float32)
        return carry

    lax.fori_loop(0, BATCH, zbody, 0)
    for off, sz in _VT_CHUNKS:
        pltpu.sync_copy(rows.at[pl.ds(0, sz)],
                        acc.at[pl.ds(sid * VT + off, sz)])
    plsc.subcore_barrier()

    nb = (NBATCH - wid + NW - 1) // NW

    def body(i, carry):
        base = (wid + i * NW) * BATCH
        pltpu.sync_copy(v_hbm.at[pl.ds(base, BATCH)], vbuf)
        pltpu.sync_copy(e_hbm.at[pl.ds(base, BATCH)], ebuf)
        cp = pltpu.async_copy(efea_hbm.at[ebuf], rows, sem)
        for j in range(BATCH // L):
            vi = vbuf[pl.ds(j * L, L)]
            ei = ebuf[pl.ds(j * L, L)]
            w = _edge_weights(s_loc, q_loc, vi, ei)
            cbuf[pl.ds(j * L, L)] = w * plsc.load_gather(rden, [vi])
        cp.wait()

        def scale(r, carry2):
            cv = plsc.load_gather(cbuf, [jnp.full((L,), 0, jnp.int32) + r])
            for ch in range(D // L):
                rows[r, pl.ds(ch * L, L)] = rows[r, pl.ds(ch * L, L)] * cv
            return carry2

        lax.fori_loop(0, BATCH, scale, 0)
        pltpu.sync_copy(rows, acc.at[vbuf], add=True)
        return carry

    lax.fori_loop(0, nb, body, 0)
    plsc.subcore_barrier()
    for off, sz in _VT_CHUNKS:
        pltpu.sync_copy(acc.at[pl.ds(sid * VT + off, sz)],
                        rows.at[pl.ds(0, sz)])
        pltpu.sync_copy(rows.at[pl.ds(0, sz)],
                        part_hbm.at[cid, pl.ds(sid * VT + off, sz)])


def _combine_body(p_ref, o_ref):
    o_ref[...] = p_ref[0] + p_ref[1]


_combine = pl.pallas_call(
    _combine_body,
    grid=(V_PAD // 128,),
    in_specs=[pl.BlockSpec((2, 128, D), lambda i: (0, i, 0))],
    out_specs=pl.BlockSpec((128, D), lambda i: (i, 0)),
    out_shape=jax.ShapeDtypeStruct((V_PAD, D), jnp.float32))


def kernel(v_fea, t_fea, e_fea, ve, a_v_w, a_t_w, a_e_w):
    ve = ve.astype(jnp.int32)
    v_idx = ve[:, 0]
    e_idx = ve[:, 1]
    s2, q2 = _scores(v_fea, t_fea, e_fea, a_v_w, a_t_w, a_e_w)
    s = s2.reshape(V_NUM)
    q = q2.reshape(E_NUM)
    den0, den1 = _denom_kernel(s, q, v_idx, e_idx)
    part = _agg_kernel(s, q, v_idx, e_idx, den0, den1, e_fea)
    return _combine(part)[:V_NUM]


# speedup vs baseline: 33.3929x; 1.5260x over previous
import functools

import jax
import jax.numpy as jnp
from jax import lax
from jax.experimental import pallas as pl
from jax.experimental.pallas import tpu as pltpu
from jax.experimental.pallas import tpu_sc as plsc

V_NUM = 10000
E_NUM = 10000
N_INC = 320000
D = 128
L = 16
NC = 2
NS = 16
NW = NC * NS
BATCH = 128
NB_TILE = 78
NPAIR = NB_TILE // 2
XBASE = 2496
VT = 632
V_PAD = VT * NS
IC_TILE = N_INC // NW
CB = 2000
NCHUNK = IC_TILE // CB

_MESH = plsc.VectorSubcoreMesh(
    core_axis_name="c", subcore_axis_name="s", num_cores=NC, num_subcores=NS)
_SC_PARAMS = pltpu.CompilerParams(needs_layout_passes=False)


def _scores_body(v_ref, t_ref, e_ref, wv_ref, wt_ref, we_ref, s_ref, q_ref):
    s_ref[...] = (
        jnp.dot(v_ref[...], wv_ref[...], preferred_element_type=jnp.float32)
        + jnp.dot(t_ref[...], wt_ref[...], preferred_element_type=jnp.float32))
    q_ref[...] = jnp.dot(e_ref[...], we_ref[...],
                         preferred_element_type=jnp.float32)


_scores = pl.pallas_call(
    _scores_body,
    out_shape=(jax.ShapeDtypeStruct((V_NUM, 1), jnp.float32),
               jax.ShapeDtypeStruct((E_NUM, 1), jnp.float32)))


@functools.partial(
    pl.kernel,
    out_type=(jax.ShapeDtypeStruct((V_PAD,), jnp.float32),
              jax.ShapeDtypeStruct((V_PAD,), jnp.float32),
              jax.ShapeDtypeStruct((N_INC,), jnp.float32)),
    mesh=_MESH,
    compiler_params=_SC_PARAMS,
    scratch_types=[
        pltpu.VMEM((V_NUM,), jnp.float32),
        pltpu.VMEM((E_NUM,), jnp.float32),
        pltpu.VMEM((CB,), jnp.int32),
        pltpu.VMEM((CB,), jnp.int32),
        pltpu.VMEM((CB,), jnp.float32),
        pltpu.VMEM((640,), jnp.float32),
        pltpu.VMEM_SHARED((V_PAD,), jnp.float32),
    ],
)
def _denom_kernel(s_hbm, q_hbm, v_hbm, e_hbm, den0_hbm, den1_hbm, w_hbm,
                  s_loc, q_loc, vbuf, ebuf, wbuf, zbuf, dacc):
    cid = lax.axis_index("c")
    sid = lax.axis_index("s")
    wid = cid * NS + sid
    pltpu.sync_copy(s_hbm, s_loc)
    pltpu.sync_copy(q_hbm, q_loc)

    def zbody(i, carry):
        zbuf[pl.ds(i * L, L)] = jnp.zeros((L,), jnp.float32)
        return carry

    lax.fori_loop(0, 640 // L, zbody, 0)
    pltpu.sync_copy(zbuf.at[pl.ds(0, VT)], dacc.at[pl.ds(sid * VT, VT)])
    plsc.subcore_barrier()

    def body(c, carry):
        base = wid * IC_TILE + c * CB
        pltpu.sync_copy(v_hbm.at[pl.ds(base, CB)], vbuf)
        pltpu.sync_copy(e_hbm.at[pl.ds(base, CB)], ebuf)

        def grp(j, carry2):
            vi = vbuf[pl.ds(j * L, L)]
            ei = ebuf[pl.ds(j * L, L)]
            x = plsc.load_gather(s_loc, [vi]) + plsc.load_gather(q_loc, [ei])
            t = jnp.exp(x + x)
            th = 1.0 - 2.0 / (t + 1.0)
            wbuf[pl.ds(j * L, L)] = jnp.exp(th + th)
            return carry2

        lax.fori_loop(0, CB // L, grp, 0)
        pltpu.sync_copy(wbuf, w_hbm.at[pl.ds(base, CB)])
        pltpu.sync_copy(wbuf, dacc.at[vbuf], add=True)
        return carry

    lax.fori_loop(0, NCHUNK, body, 0)
    plsc.subcore_barrier()

    pltpu.sync_copy(dacc.at[pl.ds(sid * VT, VT)], zbuf.at[pl.ds(0, VT)])

    @pl.when(cid == 0)
    def _():
        pltpu.sync_copy(zbuf.at[pl.ds(0, VT)], den0_hbm.at[pl.ds(sid * VT, VT)])

    @pl.when(cid == 1)
    def _():
        pltpu.sync_copy(zbuf.at[pl.ds(0, VT)], den1_hbm.at[pl.ds(sid * VT, VT)])


_VT_CHUNKS = ((0, 128), (128, 128), (256, 128), (384, 128), (512, 120))


@functools.partial(
    pl.kernel,
    out_type=jax.ShapeDtypeStruct((NC, V_PAD, D), jnp.float32),
    mesh=_MESH,
    compiler_params=_SC_PARAMS,
    scratch_types=[
        pltpu.VMEM((V_PAD,), jnp.float32),
        pltpu.VMEM((BATCH,), jnp.int32),
        pltpu.VMEM((BATCH,), jnp.int32),
        pltpu.VMEM((BATCH,), jnp.int32),
        pltpu.VMEM((BATCH,), jnp.int32),
        pltpu.VMEM((BATCH,), jnp.float32),
        pltpu.VMEM((BATCH,), jnp.float32),
        pltpu.VMEM((BATCH,), jnp.float32),
        pltpu.VMEM((BATCH, D), jnp.float32),
        pltpu.VMEM((BATCH, D), jnp.float32),
        pltpu.VMEM_SHARED((V_PAD, D), jnp.float32),
        pltpu.SemaphoreType.DMA,
        pltpu.SemaphoreType.DMA,
        pltpu.SemaphoreType.DMA,
    ],
)
def _agg_kernel(v_hbm, e_hbm, w_hbm, den0_hbm, den1_hbm, efea_hbm,
                part_hbm,
                rden, vb0, vb1, eb0, eb1, wv0, wv1, cbuf, rows0, rows1, acc,
                isem, gsem, ssem):
    cid = lax.axis_index("c")
    sid = lax.axis_index("s")
    wid = cid * NS + sid

    pltpu.sync_copy(den0_hbm, rden)

    def rbody(i, carry):
        pltpu.sync_copy(den1_hbm.at[pl.ds(i * BATCH, BATCH)], cbuf)
        for j in range(BATCH // L):
            sl = pl.ds(i * BATCH + j * L, L)
            rden[sl] = 1.0 / (rden[sl] + cbuf[pl.ds(j * L, L)])
        return carry

    lax.fori_loop(0, V_PAD // BATCH, rbody, 0)

    def zbody(r, carry):
        for ch in range(D // L):
            rows0[r, pl.ds(ch * L, L)] = jnp.zeros((L,), jnp.float32)
        return carry

    lax.fori_loop(0, BATCH, zbody, 0)
    for off, sz in _VT_CHUNKS:
        pltpu.sync_copy(rows0.at[pl.ds(0, sz)],
                        acc.at[pl.ds(sid * VT + off, sz)])
    plsc.subcore_barrier()

    def issue_idx(k, vb, eb, wv):
        base = k * BATCH
        pltpu.make_async_copy(v_hbm.at[pl.ds(base, BATCH)], vb, isem).start()
        pltpu.make_async_copy(e_hbm.at[pl.ds(base, BATCH)], eb, isem).start()
        pltpu.make_async_copy(w_hbm.at[pl.ds(base, BATCH)], wv, isem).start()

    def wait_idx(vb, eb, wv):
        pltpu.make_async_copy(v_hbm.at[pl.ds(0, BATCH)], vb, isem).wait()
        pltpu.make_async_copy(e_hbm.at[pl.ds(0, BATCH)], eb, isem).wait()
        pltpu.make_async_copy(w_hbm.at[pl.ds(0, BATCH)], wv, isem).wait()

    def start_gather(eb, rows):
        pltpu.make_async_copy(efea_hbm.at[eb], rows, gsem).start()

    def wait_gather(eb, rows):
        pltpu.make_async_copy(efea_hbm.at[eb], rows, gsem).wait()

    def start_scatter(rows, vb):
        pltpu.make_async_copy(rows, acc.at[vb], ssem).start(add=True)

    def wait_scatter(rows, vb):
        pltpu.make_async_copy(rows, acc.at[vb], ssem).wait()

    def scale_rows(rows, vb, wv):
        for j in range(BATCH // L):
            sl = pl.ds(j * L, L)
            cbuf[sl] = wv[sl] * plsc.load_gather(rden, [vb[sl]])

        def sbody(rr, carry):
            for u in range(4):
                r = rr * 4 + u
                cv = plsc.load_gather(cbuf, [jnp.full((L,), 0, jnp.int32) + r])
                for ch in range(D // L):
                    rows[r, pl.ds(ch * L, L)] = rows[r, pl.ds(ch * L, L)] * cv
            return carry

        lax.fori_loop(0, BATCH // 4, sbody, 0)

    b0 = wid * NB_TILE
    pltpu.sync_copy(v_hbm.at[pl.ds(b0 * BATCH, BATCH)], vb0)
    pltpu.sync_copy(e_hbm.at[pl.ds(b0 * BATCH, BATCH)], eb0)
    pltpu.sync_copy(w_hbm.at[pl.ds(b0 * BATCH, BATCH)], wv0)
    start_gather(eb0, rows0)

    def pair(kk, carry):
        k0 = b0 + 2 * kk
        wait_gather(eb0, rows0)

        @pl.when(kk > 0)
        def _():
            wait_scatter(rows1, vb1)

        issue_idx(k0 + 1, vb1, eb1, wv1)
        wait_idx(vb1, eb1, wv1)
        start_gather(eb1, rows1)
        scale_rows(rows0, vb0, wv0)
        start_scatter(rows0, vb0)
        wait_gather(eb1, rows1)
        wait_scatter(rows0, vb0)

        @pl.when(kk < NPAIR - 1)
        def _():
            issue_idx(k0 + 2, vb0, eb0, wv0)
            wait_idx(vb0, eb0, wv0)
            start_gather(eb0, rows0)

        scale_rows(rows1, vb1, wv1)
        start_scatter(rows1, vb1)
        return carry

    lax.fori_loop(0, NPAIR, pair, 0)
    wait_scatter(rows1, vb1)

    @pl.when(wid < 4)
    def _():
        bx = (XBASE + wid) * BATCH
        pltpu.sync_copy(v_hbm.at[pl.ds(bx, BATCH)], vb0)
        pltpu.sync_copy(e_hbm.at[pl.ds(bx, BATCH)], eb0)
        pltpu.sync_copy(w_hbm.at[pl.ds(bx, BATCH)], wv0)
        start_gather(eb0, rows0)
        wait_gather(eb0, rows0)
        scale_rows(rows0, vb0, wv0)
        pltpu.sync_copy(rows0, acc.at[vb0], add=True)

    plsc.subcore_barrier()
    for off, sz in _VT_CHUNKS:
        pltpu.sync_copy(acc.at[pl.ds(sid * VT + off, sz)],
                        rows0.at[pl.ds(0, sz)])
        pltpu.sync_copy(rows0.at[pl.ds(0, sz)],
                        part_hbm.at[cid, pl.ds(sid * VT + off, sz)])


def _combine_body(p_ref, o_ref):
    o_ref[...] = p_ref[0] + p_ref[1]


_combine = pl.pallas_call(
    _combine_body,
    grid=(V_PAD // 128,),
    in_specs=[pl.BlockSpec((2, 128, D), lambda i: (0, i, 0))],
    out_specs=pl.BlockSpec((128, D), lambda i: (i, 0)),
    out_shape=jax.ShapeDtypeStruct((V_PAD, D), jnp.float32))


def kernel(v_fea, t_fea, e_fea, ve, a_v_w, a_t_w, a_e_w):
    ve = ve.astype(jnp.int32)
    v_idx = ve[:, 0]
    e_idx = ve[:, 1]
    s2, q2 = _scores(v_fea, t_fea, e_fea, a_v_w, a_t_w, a_e_w)
    s = s2.reshape(V_NUM)
    q = q2.reshape(E_NUM)
    den0, den1, w = _denom_kernel(s, q, v_idx, e_idx)
    part = _agg_kernel(v_idx, e_idx, w, den0, den1, e_fea)
    return _combine(part)[:V_NUM]
